# Initial kernel scaffold; baseline (speedup 1.0000x reference)
import jax
import jax.numpy as jnp
from jax.experimental import pallas as pl


def kernel(x, edge_index, batch, W1r, W1s, b1, W2r, W2s, b2, W3r, W3s, b3, Wlin, blin):
    raise NotImplementedError("write your pallas kernel here")


# speedup vs baseline: 4.7508x; 4.7508x over previous
import functools

import jax
import jax.numpy as jnp
from jax import lax
from jax.experimental import pallas as pl
from jax.experimental.pallas import tpu as pltpu
from jax.experimental.pallas import tpu_sc as plsc

N = 10000
E = 320000
D = 128
G = 64

NC = 2
NS = 16
NW = NC * NS
EPW = E // NW
EPB = 80
RPS = 624
REM = N - NS * RPS
ZR = 24


def _seg_sum_kernel(x_hbm, ei_hbm, out_hbm, agg_s, src_v, dst_v, rows_v,
                    zero_v, sem):
    c = lax.axis_index("c")
    s = lax.axis_index("s")
    wid = s * NC + c

    @pl.loop(0, ZR)
    def _(r):
        @pl.loop(0, D, step=16)
        def _(cc):
            zero_v[r, pl.ds(cc, 16)] = jnp.zeros((16,), jnp.float32)

    @pl.loop(0, RPS, step=ZR)
    def _(j):
        pltpu.sync_copy(zero_v, agg_s.at[pl.ds(s * RPS + j, ZR)])

    @pl.when(s == 0)
    def _():
        pltpu.sync_copy(zero_v.at[pl.ds(0, REM)],
                        agg_s.at[pl.ds(NS * RPS, REM)])

    plsc.subcore_barrier()

    base0 = wid * EPW

    @pl.loop(0, EPW, step=EPB)
    def _(e):
        b = base0 + e
        pltpu.sync_copy(ei_hbm.at[pl.ds(b, EPB)], src_v)
        pltpu.sync_copy(ei_hbm.at[pl.ds(E + b, EPB)], dst_v)
        pltpu.async_copy(x_hbm.at[src_v], rows_v, sem).wait()
        pltpu.sync_copy(rows_v, agg_s.at[dst_v], add=True)

    plsc.subcore_barrier()

    pltpu.sync_copy(agg_s.at[pl.ds(s * RPS, RPS)],
                    out_hbm.at[c, pl.ds(s * RPS, RPS)])

    @pl.when(s == 0)
    def _():
        pltpu.sync_copy(agg_s.at[pl.ds(NS * RPS, REM)],
                        out_hbm.at[c, pl.ds(NS * RPS, REM)])


def _seg_sum(x):
    mesh = plsc.VectorSubcoreMesh(core_axis_name="c", subcore_axis_name="s")
    return functools.partial(
        pl.kernel,
        mesh=mesh,
        out_type=jax.ShapeDtypeStruct((NC, N, D), jnp.float32),
        scratch_types=[
            pltpu.VMEM_SHARED((N, D), jnp.float32),
            pltpu.VMEM((EPB,), jnp.int32),
            pltpu.VMEM((EPB,), jnp.int32),
            pltpu.VMEM((EPB, D), jnp.float32),
            pltpu.VMEM((ZR, D), jnp.float32),
            pltpu.SemaphoreType.DMA,
        ],
    )(_seg_sum_kernel)(x[0], x[1].reshape(-1))


BLK = 1000


def _layer_body(agg_ref, x_ref, wr_ref, ws_ref, b_ref, o_ref, *, relu):
    a = agg_ref[0] + agg_ref[1]
    acc = jnp.dot(a, wr_ref[...], preferred_element_type=jnp.float32)
    acc = acc + jnp.dot(x_ref[...], ws_ref[...],
                        preferred_element_type=jnp.float32)
    acc = acc + b_ref[...]
    o_ref[...] = jnp.maximum(acc, 0.0) if relu else acc


def _tc_layer(agg, x, wr, ws, b, relu):
    return pl.pallas_call(
        functools.partial(_layer_body, relu=relu),
        grid=(N // BLK,),
        in_specs=[
            pl.BlockSpec((NC, BLK, D), lambda i: (0, i, 0)),
            pl.BlockSpec((BLK, D), lambda i: (i, 0)),
            pl.BlockSpec((D, D), lambda i: (0, 0)),
            pl.BlockSpec((D, D), lambda i: (0, 0)),
            pl.BlockSpec((1, D), lambda i: (0, 0)),
        ],
        out_specs=pl.BlockSpec((BLK, D), lambda i: (i, 0)),
        out_shape=jax.ShapeDtypeStruct((N, D), jnp.float32),
    )(agg, x, wr, ws, b.reshape(1, D))


def _final_body(agg_ref, x_ref, wr_ref, ws_ref, b_ref, batch_ref, wlin_ref,
                blin_ref, o_ref, pooled, counts):
    i = pl.program_id(0)

    @pl.when(i == 0)
    def _():
        pooled[...] = jnp.zeros_like(pooled)
        counts[...] = jnp.zeros_like(counts)

    a = agg_ref[0] + agg_ref[1]
    h = jnp.dot(a, wr_ref[...], preferred_element_type=jnp.float32)
    h = h + jnp.dot(x_ref[...], ws_ref[...],
                    preferred_element_type=jnp.float32)
    h = h + b_ref[...]

    bvec = batch_ref[...].reshape(1, BLK)
    onehot = (lax.broadcasted_iota(jnp.int32, (G, BLK), 0) == bvec)
    onehot = onehot.astype(jnp.float32)
    pooled[...] += jnp.dot(onehot, h, preferred_element_type=jnp.float32)
    counts[...] += jnp.dot(onehot, jnp.ones((BLK, D), jnp.float32),
                           preferred_element_type=jnp.float32)

    @pl.when(i == pl.num_programs(0) - 1)
    def _():
        pm = pooled[...] / jnp.maximum(counts[...], 1.0)
        o_ref[...] = jnp.dot(pm, wlin_ref[...],
                             preferred_element_type=jnp.float32) + blin_ref[...]


def _tc_final(agg, x, wr, ws, b, batch, wlin_pad, blin_pad):
    return pl.pallas_call(
        _final_body,
        grid=(N // BLK,),
        in_specs=[
            pl.BlockSpec((NC, BLK, D), lambda i: (0, i, 0)),
            pl.BlockSpec((BLK, D), lambda i: (i, 0)),
            pl.BlockSpec((D, D), lambda i: (0, 0)),
            pl.BlockSpec((D, D), lambda i: (0, 0)),
            pl.BlockSpec((1, D), lambda i: (0, 0)),
            pl.BlockSpec((1, 1, BLK), lambda i: (i, 0, 0)),
            pl.BlockSpec((D, D), lambda i: (0, 0)),
            pl.BlockSpec((1, D), lambda i: (0, 0)),
        ],
        out_specs=pl.BlockSpec((G, D), lambda i: (0, 0)),
        out_shape=jax.ShapeDtypeStruct((G, D), jnp.float32),
        scratch_shapes=[
            pltpu.VMEM((G, D), jnp.float32),
            pltpu.VMEM((G, D), jnp.float32),
        ],
    )(agg, x, wr, ws, b.reshape(1, D), batch.reshape(N // BLK, 1, BLK),
      wlin_pad, blin_pad)


def kernel(x, edge_index, batch, W1r, W1s, b1, W2r, W2s, b2, W3r, W3s, b3,
           Wlin, blin):
    C = Wlin.shape[1]
    wlin_pad = jnp.zeros((D, D), jnp.float32).at[:, :C].set(Wlin)
    blin_pad = jnp.zeros((1, D), jnp.float32).at[0, :C].set(blin)

    agg = _seg_sum((x, edge_index))
    h1 = _tc_layer(agg, x, W1r, W1s, b1, relu=True)
    agg = _seg_sum((h1, edge_index))
    h2 = _tc_layer(agg, h1, W2r, W2s, b2, relu=True)
    agg = _seg_sum((h2, edge_index))
    out = _tc_final(agg, h2, W3r, W3s, b3, batch, wlin_pad, blin_pad)
    return out[:, :C]


# speedup vs baseline: 11.4658x; 2.4134x over previous
import functools

import jax
import jax.numpy as jnp
from jax import lax
from jax.experimental import pallas as pl
from jax.experimental.pallas import tpu as pltpu
from jax.experimental.pallas import tpu_sc as plsc

N = 10000
E = 320000
D = 128
G = 64

NC = 2
NS = 16
NW = NC * NS
EPW = E // NW
EPB = 80
RPS = 624
REM = N - NS * RPS
ZR = 24


NCH = EPW // EPB


def _seg_sum_kernel(x_hbm, ei_hbm, out_hbm, agg_s, src_all, dst0, dst1,
                    rows0, rows1, zero_v, semd0, semd1, semg0, semg1):
    c = lax.axis_index("c")
    s = lax.axis_index("s")
    wid = s * NC + c

    @pl.loop(0, ZR)
    def _(r):
        @pl.loop(0, D, step=16)
        def _(cc):
            zero_v[r, pl.ds(cc, 16)] = jnp.zeros((16,), jnp.float32)

    @pl.loop(0, RPS, step=ZR)
    def _(j):
        pltpu.sync_copy(zero_v, agg_s.at[pl.ds(s * RPS + j, ZR)])

    @pl.when(s == 0)
    def _():
        pltpu.sync_copy(zero_v.at[pl.ds(0, REM)],
                        agg_s.at[pl.ds(NS * RPS, REM)])

    plsc.subcore_barrier()

    base0 = wid * EPW
    pltpu.sync_copy(ei_hbm.at[pl.ds(base0, EPW)], src_all)

    def issue(ch, dstb, rowsb, sd, sg):
        @pl.when(ch < NCH)
        def _():
            pltpu.async_copy(ei_hbm.at[pl.ds(E + base0 + ch * EPB, EPB)],
                             dstb, sd)
            pltpu.async_copy(x_hbm.at[src_all.at[pl.ds(ch * EPB, EPB)]],
                             rowsb, sg)

    def consume(ch, dstb, rowsb, sd, sg):
        @pl.when(ch < NCH)
        def _():
            pltpu.make_async_copy(
                ei_hbm.at[pl.ds(E + base0 + ch * EPB, EPB)], dstb, sd).wait()
            pltpu.make_async_copy(
                x_hbm.at[src_all.at[pl.ds(ch * EPB, EPB)]], rowsb, sg).wait()
            pltpu.sync_copy(rowsb, agg_s.at[dstb], add=True)

    issue(0, dst0, rows0, semd0, semg0)
    issue(1, dst1, rows1, semd1, semg1)

    @pl.loop(0, NCH, step=2)
    def _(j):
        consume(j, dst0, rows0, semd0, semg0)
        issue(j + 2, dst0, rows0, semd0, semg0)
        consume(j + 1, dst1, rows1, semd1, semg1)
        issue(j + 3, dst1, rows1, semd1, semg1)

    plsc.subcore_barrier()

    pltpu.sync_copy(agg_s.at[pl.ds(s * RPS, RPS)],
                    out_hbm.at[c, pl.ds(s * RPS, RPS)])

    @pl.when(s == 0)
    def _():
        pltpu.sync_copy(agg_s.at[pl.ds(NS * RPS, REM)],
                        out_hbm.at[c, pl.ds(NS * RPS, REM)])


def _seg_sum(x):
    mesh = plsc.VectorSubcoreMesh(core_axis_name="c", subcore_axis_name="s")
    return functools.partial(
        pl.kernel,
        mesh=mesh,
        out_type=jax.ShapeDtypeStruct((NC, N, D), jnp.float32),
        scratch_types=[
            pltpu.VMEM_SHARED((N, D), jnp.float32),
            pltpu.VMEM((EPW,), jnp.int32),
            pltpu.VMEM((EPB,), jnp.int32),
            pltpu.VMEM((EPB,), jnp.int32),
            pltpu.VMEM((EPB, D), jnp.float32),
            pltpu.VMEM((EPB, D), jnp.float32),
            pltpu.VMEM((ZR, D), jnp.float32),
            pltpu.SemaphoreType.DMA,
            pltpu.SemaphoreType.DMA,
            pltpu.SemaphoreType.DMA,
            pltpu.SemaphoreType.DMA,
        ],
    )(_seg_sum_kernel)(x[0], x[1].reshape(-1))


BLK = 1000


def _layer_body(agg_ref, x_ref, wr_ref, ws_ref, b_ref, o_ref, *, relu):
    a = agg_ref[0] + agg_ref[1]
    acc = jnp.dot(a, wr_ref[...], preferred_element_type=jnp.float32)
    acc = acc + jnp.dot(x_ref[...], ws_ref[...],
                        preferred_element_type=jnp.float32)
    acc = acc + b_ref[...]
    o_ref[...] = jnp.maximum(acc, 0.0) if relu else acc


def _tc_layer(agg, x, wr, ws, b, relu):
    return pl.pallas_call(
        functools.partial(_layer_body, relu=relu),
        grid=(N // BLK,),
        in_specs=[
            pl.BlockSpec((NC, BLK, D), lambda i: (0, i, 0)),
            pl.BlockSpec((BLK, D), lambda i: (i, 0)),
            pl.BlockSpec((D, D), lambda i: (0, 0)),
            pl.BlockSpec((D, D), lambda i: (0, 0)),
            pl.BlockSpec((1, D), lambda i: (0, 0)),
        ],
        out_specs=pl.BlockSpec((BLK, D), lambda i: (i, 0)),
        out_shape=jax.ShapeDtypeStruct((N, D), jnp.float32),
    )(agg, x, wr, ws, b.reshape(1, D))


def _final_body(agg_ref, x_ref, wr_ref, ws_ref, b_ref, batch_ref, wlin_ref,
                blin_ref, o_ref, pooled, counts):
    i = pl.program_id(0)

    @pl.when(i == 0)
    def _():
        pooled[...] = jnp.zeros_like(pooled)
        counts[...] = jnp.zeros_like(counts)

    a = agg_ref[0] + agg_ref[1]
    h = jnp.dot(a, wr_ref[...], preferred_element_type=jnp.float32)
    h = h + jnp.dot(x_ref[...], ws_ref[...],
                    preferred_element_type=jnp.float32)
    h = h + b_ref[...]

    bvec = batch_ref[...].reshape(1, BLK)
    onehot = (lax.broadcasted_iota(jnp.int32, (G, BLK), 0) == bvec)
    onehot = onehot.astype(jnp.float32)
    pooled[...] += jnp.dot(onehot, h, preferred_element_type=jnp.float32)
    counts[...] += jnp.dot(onehot, jnp.ones((BLK, D), jnp.float32),
                           preferred_element_type=jnp.float32)

    @pl.when(i == pl.num_programs(0) - 1)
    def _():
        pm = pooled[...] / jnp.maximum(counts[...], 1.0)
        o_ref[...] = jnp.dot(pm, wlin_ref[...],
                             preferred_element_type=jnp.float32) + blin_ref[...]


def _tc_final(agg, x, wr, ws, b, batch, wlin_pad, blin_pad):
    return pl.pallas_call(
        _final_body,
        grid=(N // BLK,),
        in_specs=[
            pl.BlockSpec((NC, BLK, D), lambda i: (0, i, 0)),
            pl.BlockSpec((BLK, D), lambda i: (i, 0)),
            pl.BlockSpec((D, D), lambda i: (0, 0)),
            pl.BlockSpec((D, D), lambda i: (0, 0)),
            pl.BlockSpec((1, D), lambda i: (0, 0)),
            pl.BlockSpec((1, 1, BLK), lambda i: (i, 0, 0)),
            pl.BlockSpec((D, D), lambda i: (0, 0)),
            pl.BlockSpec((1, D), lambda i: (0, 0)),
        ],
        out_specs=pl.BlockSpec((G, D), lambda i: (0, 0)),
        out_shape=jax.ShapeDtypeStruct((G, D), jnp.float32),
        scratch_shapes=[
            pltpu.VMEM((G, D), jnp.float32),
            pltpu.VMEM((G, D), jnp.float32),
        ],
    )(agg, x, wr, ws, b.reshape(1, D), batch.reshape(N // BLK, 1, BLK),
      wlin_pad, blin_pad)


def kernel(x, edge_index, batch, W1r, W1s, b1, W2r, W2s, b2, W3r, W3s, b3,
           Wlin, blin):
    C = Wlin.shape[1]
    wlin_pad = jnp.zeros((D, D), jnp.float32).at[:, :C].set(Wlin)
    blin_pad = jnp.zeros((1, D), jnp.float32).at[0, :C].set(blin)

    agg = _seg_sum((x, edge_index))
    h1 = _tc_layer(agg, x, W1r, W1s, b1, relu=True)
    agg = _seg_sum((h1, edge_index))
    h2 = _tc_layer(agg, h1, W2r, W2s, b2, relu=True)
    agg = _seg_sum((h2, edge_index))
    out = _tc_final(agg, h2, W3r, W3s, b3, batch, wlin_pad, blin_pad)
    return out[:, :C]


# speedup vs baseline: 11.7488x; 1.0247x over previous
import functools

import jax
import jax.numpy as jnp
from jax import lax
from jax.experimental import pallas as pl
from jax.experimental.pallas import tpu as pltpu
from jax.experimental.pallas import tpu_sc as plsc

N = 10000
E = 320000
D = 128
G = 64

NC = 2
NS = 16
NW = NC * NS
EPW = E // NW
EPB = 80
RPS = 624
REM = N - NS * RPS
ZR = 48


NCH = EPW // EPB
NBUF = 4


def _seg_sum_kernel(x_hbm, ei_hbm, out_hbm, agg_s, src_b, dst_b, rows_b,
                    semis, semid, semg, sems):
    c = lax.axis_index("c")
    s = lax.axis_index("s")
    wid = s * NC + c

    @pl.loop(0, ZR)
    def _(r):
        @pl.loop(0, D, step=16)
        def _(cc):
            rows_b[0, r, pl.ds(cc, 16)] = jnp.zeros((16,), jnp.float32)

    @pl.loop(0, RPS, step=ZR)
    def _(j):
        pltpu.sync_copy(rows_b.at[0, pl.ds(0, ZR)],
                        agg_s.at[pl.ds(s * RPS + j, ZR)])

    @pl.when(s == 0)
    def _():
        pltpu.sync_copy(rows_b.at[0, pl.ds(0, REM)],
                        agg_s.at[pl.ds(NS * RPS, REM)])

    base0 = wid * EPW

    def issue_src(ch, k):
        @pl.when(ch < NCH)
        def _():
            pltpu.async_copy(ei_hbm.at[pl.ds(base0 + ch * EPB, EPB)],
                             src_b.at[k], semis.at[k])

    def issue_dst(ch, k):
        @pl.when(ch < NCH)
        def _():
            pltpu.async_copy(ei_hbm.at[pl.ds(E + base0 + ch * EPB, EPB)],
                             dst_b.at[k], semid.at[k])

    def issue_g(ch, k):
        @pl.when(ch < NCH)
        def _():
            pltpu.make_async_copy(
                ei_hbm.at[pl.ds(base0 + ch * EPB, EPB)], src_b.at[k],
                semis.at[k]).wait()
            pltpu.async_copy(x_hbm.at[src_b.at[k]], rows_b.at[k],
                             semg.at[k])

    def issue_s(ch, k):
        @pl.when(ch < NCH)
        def _():
            pltpu.make_async_copy(
                ei_hbm.at[pl.ds(E + base0 + ch * EPB, EPB)], dst_b.at[k],
                semid.at[k]).wait()
            pltpu.make_async_copy(x_hbm.at[src_b.at[k]], rows_b.at[k],
                                  semg.at[k]).wait()
            pltpu.async_copy(rows_b.at[k], agg_s.at[dst_b.at[k]],
                             sems.at[k], add=True)

    def wait_s(ch, k):
        @pl.when(ch < NCH)
        def _():
            pltpu.make_async_copy(rows_b.at[k], agg_s.at[dst_b.at[k]],
                                  sems.at[k]).wait()

    for k in range(NBUF):
        issue_src(k, k)
        issue_dst(k, k)
    for k in range(NBUF):
        issue_g(k, k)

    plsc.subcore_barrier()

    @pl.loop(0, 128, step=NBUF)
    def _(j):
        for k in (0, 1):
            issue_s(j + k, k)
        for k in (0, 1):
            issue_src(j + NBUF + k, k)
        for k in (2, 3):
            issue_s(j + k, k)
        for k in (2, 3):
            issue_src(j + NBUF + k, k)
        for k in (0, 1):
            wait_s(j + k, k)
            issue_dst(j + NBUF + k, k)
            issue_g(j + NBUF + k, k)
        for k in (2, 3):
            wait_s(j + k, k)
            issue_dst(j + NBUF + k, k)
            issue_g(j + NBUF + k, k)

    plsc.subcore_barrier()

    pltpu.sync_copy(agg_s.at[pl.ds(s * RPS, RPS)],
                    out_hbm.at[c, pl.ds(s * RPS, RPS)])

    @pl.when(s == 0)
    def _():
        pltpu.sync_copy(agg_s.at[pl.ds(NS * RPS, REM)],
                        out_hbm.at[c, pl.ds(NS * RPS, REM)])


def _seg_sum(x):
    mesh = plsc.VectorSubcoreMesh(core_axis_name="c", subcore_axis_name="s")
    return functools.partial(
        pl.kernel,
        mesh=mesh,
        out_type=jax.ShapeDtypeStruct((NC, N, D), jnp.float32),
        scratch_types=[
            pltpu.VMEM_SHARED((N, D), jnp.float32),
            pltpu.VMEM((NBUF, EPB), jnp.int32),
            pltpu.VMEM((NBUF, EPB), jnp.int32),
            pltpu.VMEM((NBUF, EPB, D), jnp.float32),
            pltpu.SemaphoreType.DMA((NBUF,)),
            pltpu.SemaphoreType.DMA((NBUF,)),
            pltpu.SemaphoreType.DMA((NBUF,)),
            pltpu.SemaphoreType.DMA((NBUF,)),
        ],
    )(_seg_sum_kernel)(x[0], x[1].reshape(-1))


BLK = 1000


def _layer_body(agg_ref, x_ref, wr_ref, ws_ref, b_ref, o_ref, *, relu):
    a = agg_ref[0] + agg_ref[1]
    acc = jnp.dot(a, wr_ref[...], preferred_element_type=jnp.float32)
    acc = acc + jnp.dot(x_ref[...], ws_ref[...],
                        preferred_element_type=jnp.float32)
    acc = acc + b_ref[...]
    o_ref[...] = jnp.maximum(acc, 0.0) if relu else acc


def _tc_layer(agg, x, wr, ws, b, relu):
    return pl.pallas_call(
        functools.partial(_layer_body, relu=relu),
        grid=(N // BLK,),
        in_specs=[
            pl.BlockSpec((NC, BLK, D), lambda i: (0, i, 0)),
            pl.BlockSpec((BLK, D), lambda i: (i, 0)),
            pl.BlockSpec((D, D), lambda i: (0, 0)),
            pl.BlockSpec((D, D), lambda i: (0, 0)),
            pl.BlockSpec((1, D), lambda i: (0, 0)),
        ],
        out_specs=pl.BlockSpec((BLK, D), lambda i: (i, 0)),
        out_shape=jax.ShapeDtypeStruct((N, D), jnp.float32),
    )(agg, x, wr, ws, b.reshape(1, D))


def _final_body(agg_ref, x_ref, wr_ref, ws_ref, b_ref, batch_ref, wlin_ref,
                blin_ref, o_ref, pooled, counts):
    i = pl.program_id(0)

    @pl.when(i == 0)
    def _():
        pooled[...] = jnp.zeros_like(pooled)
        counts[...] = jnp.zeros_like(counts)

    a = agg_ref[0] + agg_ref[1]
    h = jnp.dot(a, wr_ref[...], preferred_element_type=jnp.float32)
    h = h + jnp.dot(x_ref[...], ws_ref[...],
                    preferred_element_type=jnp.float32)
    h = h + b_ref[...]

    bvec = batch_ref[...].reshape(1, BLK)
    onehot = (lax.broadcasted_iota(jnp.int32, (G, BLK), 0) == bvec)
    onehot = onehot.astype(jnp.float32)
    pooled[...] += jnp.dot(onehot, h, preferred_element_type=jnp.float32)
    counts[...] += jnp.dot(onehot, jnp.ones((BLK, D), jnp.float32),
                           preferred_element_type=jnp.float32)

    @pl.when(i == pl.num_programs(0) - 1)
    def _():
        pm = pooled[...] / jnp.maximum(counts[...], 1.0)
        o_ref[...] = jnp.dot(pm, wlin_ref[...],
                             preferred_element_type=jnp.float32) + blin_ref[...]


def _tc_final(agg, x, wr, ws, b, batch, wlin_pad, blin_pad):
    return pl.pallas_call(
        _final_body,
        grid=(N // BLK,),
        in_specs=[
            pl.BlockSpec((NC, BLK, D), lambda i: (0, i, 0)),
            pl.BlockSpec((BLK, D), lambda i: (i, 0)),
            pl.BlockSpec((D, D), lambda i: (0, 0)),
            pl.BlockSpec((D, D), lambda i: (0, 0)),
            pl.BlockSpec((1, D), lambda i: (0, 0)),
            pl.BlockSpec((1, 1, BLK), lambda i: (i, 0, 0)),
            pl.BlockSpec((D, D), lambda i: (0, 0)),
            pl.BlockSpec((1, D), lambda i: (0, 0)),
        ],
        out_specs=pl.BlockSpec((G, D), lambda i: (0, 0)),
        out_shape=jax.ShapeDtypeStruct((G, D), jnp.float32),
        scratch_shapes=[
            pltpu.VMEM((G, D), jnp.float32),
            pltpu.VMEM((G, D), jnp.float32),
        ],
    )(agg, x, wr, ws, b.reshape(1, D), batch.reshape(N // BLK, 1, BLK),
      wlin_pad, blin_pad)


def kernel(x, edge_index, batch, W1r, W1s, b1, W2r, W2s, b2, W3r, W3s, b3,
           Wlin, blin):
    C = Wlin.shape[1]
    wlin_pad = jnp.zeros((D, D), jnp.float32).at[:, :C].set(Wlin)
    blin_pad = jnp.zeros((1, D), jnp.float32).at[0, :C].set(blin)

    agg = _seg_sum((x, edge_index))
    h1 = _tc_layer(agg, x, W1r, W1s, b1, relu=True)
    agg = _seg_sum((h1, edge_index))
    h2 = _tc_layer(agg, h1, W2r, W2s, b2, relu=True)
    agg = _seg_sum((h2, edge_index))
    out = _tc_final(agg, h2, W3r, W3s, b3, batch, wlin_pad, blin_pad)
    return out[:, :C]


# speedup vs baseline: 12.0193x; 1.0230x over previous
import functools

import jax
import jax.numpy as jnp
from jax import lax
from jax.experimental import pallas as pl
from jax.experimental.pallas import tpu as pltpu
from jax.experimental.pallas import tpu_sc as plsc

N = 10000
E = 320000
D = 128
G = 64

NC = 2
NS = 16
NW = NC * NS
EPW = E // NW
EPB = 80
RPS = 624
REM = N - NS * RPS
ZR = 48


NCH = EPW // EPB
NBUF = 4


def _seg_sum_kernel(x_hbm, ei_hbm, out_hbm, agg_s, src_b, dst_b, rows_b,
                    semis, semid, semg, sems):
    c = lax.axis_index("c")
    s = lax.axis_index("s")
    wid = s * NC + c

    @pl.loop(0, ZR)
    def _(r):
        @pl.loop(0, D, step=16)
        def _(cc):
            rows_b[0, r, pl.ds(cc, 16)] = jnp.zeros((16,), jnp.float32)

    @pl.loop(0, RPS, step=ZR)
    def _(j):
        pltpu.sync_copy(rows_b.at[0, pl.ds(0, ZR)],
                        agg_s.at[pl.ds(s * RPS + j, ZR)])

    @pl.when(s == 0)
    def _():
        pltpu.sync_copy(rows_b.at[0, pl.ds(0, REM)],
                        agg_s.at[pl.ds(NS * RPS, REM)])

    base0 = wid * EPW

    def issue_src(ch, k):
        @pl.when(ch < NCH)
        def _():
            pltpu.async_copy(ei_hbm.at[pl.ds(base0 + ch * EPB, EPB)],
                             src_b.at[k], semis.at[k])

    def issue_dst(ch, k):
        @pl.when(ch < NCH)
        def _():
            pltpu.async_copy(ei_hbm.at[pl.ds(E + base0 + ch * EPB, EPB)],
                             dst_b.at[k], semid.at[k])

    def issue_g(ch, k):
        @pl.when(ch < NCH)
        def _():
            pltpu.make_async_copy(
                ei_hbm.at[pl.ds(base0 + ch * EPB, EPB)], src_b.at[k],
                semis.at[k]).wait()
            pltpu.async_copy(x_hbm.at[src_b.at[k]], rows_b.at[k],
                             semg.at[k])

    def issue_s(ch, k):
        @pl.when(ch < NCH)
        def _():
            pltpu.make_async_copy(
                ei_hbm.at[pl.ds(E + base0 + ch * EPB, EPB)], dst_b.at[k],
                semid.at[k]).wait()
            pltpu.make_async_copy(x_hbm.at[src_b.at[k]], rows_b.at[k],
                                  semg.at[k]).wait()
            pltpu.async_copy(rows_b.at[k], agg_s.at[dst_b.at[k]],
                             sems.at[k], add=True)

    def wait_s(ch, k):
        @pl.when(ch < NCH)
        def _():
            pltpu.make_async_copy(rows_b.at[k], agg_s.at[dst_b.at[k]],
                                  sems.at[k]).wait()

    for k in range(NBUF):
        issue_src(k, k)
        issue_dst(k, k)
    for k in range(NBUF):
        issue_g(k, k)

    plsc.subcore_barrier()

    @pl.loop(0, 128, step=NBUF)
    def _(j):
        for k in (0, 1):
            issue_s(j + k, k)
        for k in (0, 1):
            issue_src(j + NBUF + k, k)
        for k in (2, 3):
            issue_s(j + k, k)
        for k in (2, 3):
            issue_src(j + NBUF + k, k)
        for k in (0, 1):
            wait_s(j + k, k)
            issue_dst(j + NBUF + k, k)
            issue_g(j + NBUF + k, k)
        for k in (2, 3):
            wait_s(j + k, k)
            issue_dst(j + NBUF + k, k)
            issue_g(j + NBUF + k, k)

    plsc.subcore_barrier()

    pltpu.sync_copy(agg_s.at[pl.ds(s * RPS, RPS)],
                    out_hbm.at[c, pl.ds(s * RPS, RPS)])

    @pl.when(s == 0)
    def _():
        pltpu.sync_copy(agg_s.at[pl.ds(NS * RPS, REM)],
                        out_hbm.at[c, pl.ds(NS * RPS, REM)])


def _seg_sum(x):
    mesh = plsc.VectorSubcoreMesh(core_axis_name="c", subcore_axis_name="s")
    return functools.partial(
        pl.kernel,
        mesh=mesh,
        out_type=jax.ShapeDtypeStruct((NC, N, D), jnp.float32),
        scratch_types=[
            pltpu.VMEM_SHARED((N, D), jnp.float32),
            pltpu.VMEM((NBUF, EPB), jnp.int32),
            pltpu.VMEM((NBUF, EPB), jnp.int32),
            pltpu.VMEM((NBUF, EPB, D), jnp.float32),
            pltpu.SemaphoreType.DMA((NBUF,)),
            pltpu.SemaphoreType.DMA((NBUF,)),
            pltpu.SemaphoreType.DMA((NBUF,)),
            pltpu.SemaphoreType.DMA((NBUF,)),
        ],
    )(_seg_sum_kernel)(x[0], x[1].reshape(-1))


BLK = 2000


def _pre_body(x_ref, ws_ref, b_ref, o_ref):
    o_ref[...] = jnp.dot(x_ref[...], ws_ref[...],
                         preferred_element_type=jnp.float32) + b_ref[...]


def _tc_pre(x, ws, b):
    return pl.pallas_call(
        _pre_body,
        grid=(N // BLK,),
        in_specs=[
            pl.BlockSpec((BLK, D), lambda i: (i, 0)),
            pl.BlockSpec((D, D), lambda i: (0, 0)),
            pl.BlockSpec((1, D), lambda i: (0, 0)),
        ],
        out_specs=pl.BlockSpec((BLK, D), lambda i: (i, 0)),
        out_shape=jax.ShapeDtypeStruct((N, D), jnp.float32),
    )(x, ws, b.reshape(1, D))


def _post_body(agg_ref, xs_ref, wr_ref, o_ref):
    a = agg_ref[0] + agg_ref[1]
    acc = jnp.dot(a, wr_ref[...], preferred_element_type=jnp.float32)
    o_ref[...] = jnp.maximum(acc + xs_ref[...], 0.0)


def _tc_post(agg, xs, wr):
    return pl.pallas_call(
        _post_body,
        grid=(N // BLK,),
        in_specs=[
            pl.BlockSpec((NC, BLK, D), lambda i: (0, i, 0)),
            pl.BlockSpec((BLK, D), lambda i: (i, 0)),
            pl.BlockSpec((D, D), lambda i: (0, 0)),
        ],
        out_specs=pl.BlockSpec((BLK, D), lambda i: (i, 0)),
        out_shape=jax.ShapeDtypeStruct((N, D), jnp.float32),
    )(agg, xs, wr)


def _final_body(agg_ref, xs_ref, wr_ref, batch_ref, wlin_ref, blin_ref,
                o_ref, pooled, counts):
    i = pl.program_id(0)

    @pl.when(i == 0)
    def _():
        pooled[...] = jnp.zeros_like(pooled)
        counts[...] = jnp.zeros_like(counts)

    a = agg_ref[0] + agg_ref[1]
    h = jnp.dot(a, wr_ref[...], preferred_element_type=jnp.float32)
    h = h + xs_ref[...]

    bvec = batch_ref[...].reshape(1, BLK)
    onehot = (lax.broadcasted_iota(jnp.int32, (G, BLK), 0) == bvec)
    onehot = onehot.astype(jnp.float32)
    pooled[...] += jnp.dot(onehot, h, preferred_element_type=jnp.float32)
    counts[...] += jnp.dot(onehot, jnp.ones((BLK, D), jnp.float32),
                           preferred_element_type=jnp.float32)

    @pl.when(i == pl.num_programs(0) - 1)
    def _():
        pm = pooled[...] / jnp.maximum(counts[...], 1.0)
        o_ref[...] = jnp.dot(pm, wlin_ref[...],
                             preferred_element_type=jnp.float32) + blin_ref[...]


def _tc_final(agg, xs, wr, batch, wlin_pad, blin_pad):
    return pl.pallas_call(
        _final_body,
        grid=(N // BLK,),
        in_specs=[
            pl.BlockSpec((NC, BLK, D), lambda i: (0, i, 0)),
            pl.BlockSpec((BLK, D), lambda i: (i, 0)),
            pl.BlockSpec((D, D), lambda i: (0, 0)),
            pl.BlockSpec((1, 1, BLK), lambda i: (i, 0, 0)),
            pl.BlockSpec((D, D), lambda i: (0, 0)),
            pl.BlockSpec((1, D), lambda i: (0, 0)),
        ],
        out_specs=pl.BlockSpec((G, D), lambda i: (0, 0)),
        out_shape=jax.ShapeDtypeStruct((G, D), jnp.float32),
        scratch_shapes=[
            pltpu.VMEM((G, D), jnp.float32),
            pltpu.VMEM((G, D), jnp.float32),
        ],
    )(agg, xs, wr, batch.reshape(N // BLK, 1, BLK), wlin_pad, blin_pad)


def kernel(x, edge_index, batch, W1r, W1s, b1, W2r, W2s, b2, W3r, W3s, b3,
           Wlin, blin):
    C = Wlin.shape[1]
    wlin_pad = jnp.zeros((D, D), jnp.float32).at[:, :C].set(Wlin)
    blin_pad = jnp.zeros((1, D), jnp.float32).at[0, :C].set(blin)

    xs1 = _tc_pre(x, W1s, b1)
    agg = _seg_sum((x, edge_index))
    h1 = _tc_post(agg, xs1, W1r)
    xs2 = _tc_pre(h1, W2s, b2)
    agg = _seg_sum((h1, edge_index))
    h2 = _tc_post(agg, xs2, W2r)
    xs3 = _tc_pre(h2, W3s, b3)
    agg = _seg_sum((h2, edge_index))
    out = _tc_final(agg, xs3, W3r, batch, wlin_pad, blin_pad)
    return out[:, :C]
